# baseline (speedup 1.0000x reference)
import functools

import jax
import jax.numpy as jnp
from jax import lax
from jax.experimental import pallas as pl
from jax.experimental.pallas import tpu as pltpu
from jax.experimental.pallas import tpu_sc as plsc

_SIDE_NUM = 1000
_VEC_LEN = 32
_B = 16384

_NC = 2
_NS = 16
_IPT = _B // _NS
_QSTEPS = _IPT // 16
_NCHUNK = _B // 128


def _quantize(v):
    v = v + jnp.float32(50.0)
    v = jnp.maximum(v, jnp.float32(1e-8))
    v = jnp.minimum(v, jnp.float32(100.0))
    p = v * jnp.float32(999.0 / 100.0)
    i0 = p.astype(jnp.int32)
    frac = p - i0.astype(jnp.float32)
    half = jnp.float32(0.5)
    odd = (i0 & 1) == 1
    inc = (frac > half) | ((frac == half) & odd)
    return jnp.where(inc, i0 + 1, i0)


def _body(xs_hbm, ys_hbm, embT_hbm, outT_hbm,
          xs_v, ys_v, ids8_v, idsv2, gath_v, ids_sh, sem):
    c = lax.axis_index("c")
    s = lax.axis_index("s")
    d = c * _NS + s

    pltpu.sync_copy(xs_hbm.at[pl.ds(_IPT * s, _IPT)], xs_v)
    pltpu.sync_copy(ys_hbm.at[pl.ds(_IPT * s, _IPT)], ys_v)

    for j in range(_QSTEPS):
        x = xs_v[pl.ds(j * 16, 16)]
        y = ys_v[pl.ds(j * 16, 16)]
        ids8_v[j // 8, pl.ds((j % 8) * 16, 16)] = (
            _quantize(x) * _SIDE_NUM + _quantize(y))

    pltpu.sync_copy(ids8_v, ids_sh.at[pl.ds(8 * s, 8)])
    plsc.subcore_barrier()
    pltpu.sync_copy(ids_sh, idsv2)

    row = embT_hbm.at[d]

    def group(g, carry):
        for k in range(8):
            pltpu.async_copy(row.at[idsv2.at[g * 8 + k]],
                             gath_v.at[g * 8 + k], sem)
        for k in range(8):
            pltpu.make_async_copy(row.at[idsv2.at[g * 8 + k]],
                                  gath_v.at[g * 8 + k], sem).wait()
        return carry

    lax.fori_loop(0, _NCHUNK // 8, group, 0)

    pltpu.sync_copy(gath_v, outT_hbm.at[d])


@jax.jit
def kernel(locations, pos_emb):
    xs = locations[:, 0]
    ys = locations[:, 1]
    emb_t = pos_emb.T
    mesh = plsc.VectorSubcoreMesh(core_axis_name="c", subcore_axis_name="s")
    run = pl.kernel(
        _body,
        mesh=mesh,
        out_type=jax.ShapeDtypeStruct((_VEC_LEN, 128, 128), jnp.float32),
        scratch_types=[
            pltpu.VMEM((_IPT,), jnp.float32),
            pltpu.VMEM((_IPT,), jnp.float32),
            pltpu.VMEM((8, 128), jnp.int32),
            pltpu.VMEM((128, 128), jnp.int32),
            pltpu.VMEM((128, 128), jnp.float32),
            pltpu.VMEM_SHARED((128, 128), jnp.int32),
            pltpu.SemaphoreType.DMA,
        ],
        compiler_params=pltpu.CompilerParams(use_tc_tiling_on_sc=False),
    )
    out_t = run(xs, ys, emb_t)
    return out_t.reshape(_VEC_LEN, _B).T


# speedup vs baseline: 4.2936x; 4.2936x over previous
import functools

import jax
import jax.numpy as jnp
from jax import lax
from jax.experimental import pallas as pl
from jax.experimental.pallas import tpu as pltpu
from jax.experimental.pallas import tpu_sc as plsc

_SIDE_NUM = 1000
_VEC_LEN = 32
_B = 16384

_NC = 2
_NS = 16
_NW = _NC * _NS
_BPW = _B // _NW
_GCH = 64
_NCH = _BPW // _GCH


def _quantize(v):
    v = v + jnp.float32(50.0)
    v = jnp.maximum(v, jnp.float32(1e-8))
    v = jnp.minimum(v, jnp.float32(100.0))
    p = v * jnp.float32(999.0 / 100.0)
    i0 = p.astype(jnp.int32)
    frac = p - i0.astype(jnp.float32)
    half = jnp.float32(0.5)
    odd = (i0 & 1) == 1
    inc = (frac > half) | ((frac == half) & odd)
    return jnp.where(inc, i0 + 1, i0)


def _body(xs_hbm, ys_hbm, table_hbm, out_hbm,
          xs_v, ys_v, ids_v, tile_v, sub_v, lan_v, rows_v, sem):
    wid = lax.axis_index("s") * _NC + lax.axis_index("c")
    base = wid * _BPW

    pltpu.sync_copy(xs_hbm.at[pl.ds(base, _BPW)], xs_v)
    pltpu.sync_copy(ys_hbm.at[pl.ds(base, _BPW)], ys_v)

    for j in range(_BPW // 16):
        x = xs_v[pl.ds(j * 16, 16)]
        y = ys_v[pl.ds(j * 16, 16)]
        ids = _quantize(x) * _SIDE_NUM + _quantize(y)
        ids_v[pl.ds(j * 16, 16)] = ids >> 5
        sub_v[pl.ds(j * 16, 16)] = (ids >> 2) & 7
        lan_v[pl.ds(j * 16, 16)] = (ids & 3) * 32

    for c in range(_NCH):
        cp = pltpu.async_copy(
            table_hbm.at[ids_v.at[pl.ds(c * _GCH, _GCH)]],
            tile_v, sem)
        cp.wait()
        for k in range(_GCH):
            r = c * _GCH + k
            subs = sub_v[pl.ds((r // 16) * 16, 16)]
            lans = lan_v[pl.ds((r // 16) * 16, 16)]
            sub = subs[r % 16]
            lan = lans[r % 16]
            col = (r % 4) * 32
            rows_v[r // 4, pl.ds(col, 16)] = tile_v[k, sub, pl.ds(lan, 16)]
            rows_v[r // 4, pl.ds(col + 16, 16)] = tile_v[k, sub, pl.ds(lan + 16, 16)]

    pltpu.sync_copy(rows_v, out_hbm.at[pl.ds(wid * 128, 128)])


@jax.jit
def kernel(locations, pos_emb):
    xs = locations[:, 0]
    ys = locations[:, 1]
    table3 = pos_emb.reshape(_SIDE_NUM ** 2 // 32, 8, 128)
    mesh = plsc.VectorSubcoreMesh(core_axis_name="c", subcore_axis_name="s")
    run = pl.kernel(
        _body,
        mesh=mesh,
        out_type=jax.ShapeDtypeStruct((_B * _VEC_LEN // 128, 128), jnp.float32),
        scratch_types=[
            pltpu.VMEM((_BPW,), jnp.float32),
            pltpu.VMEM((_BPW,), jnp.float32),
            pltpu.VMEM((_BPW,), jnp.int32),
            pltpu.VMEM((_GCH, 8, 128), jnp.float32),
            pltpu.VMEM((_BPW,), jnp.int32),
            pltpu.VMEM((_BPW,), jnp.int32),
            pltpu.VMEM((128, 128), jnp.float32),
            pltpu.SemaphoreType.DMA,
        ],
        compiler_params=pltpu.CompilerParams(use_tc_tiling_on_sc=True),
    )
    return run(xs, ys, table3).reshape(_B, _VEC_LEN)


# speedup vs baseline: 4.9449x; 1.1517x over previous
import functools

import jax
import jax.numpy as jnp
from jax import lax
from jax.experimental import pallas as pl
from jax.experimental.pallas import tpu as pltpu
from jax.experimental.pallas import tpu_sc as plsc

_SIDE_NUM = 1000
_VEC_LEN = 32
_B = 16384

_NC = 2
_NS = 16
_NW = _NC * _NS
_BPW = _B // _NW
_CHUNK = 128
_NCHUNK = _BPW // _CHUNK
_QSTEPS = _BPW // 16


def _quantize(v):
    v = v + jnp.float32(50.0)
    v = jnp.maximum(v, jnp.float32(1e-8))
    v = jnp.minimum(v, jnp.float32(100.0))
    p = v * jnp.float32(999.0 / 100.0)
    i0 = p.astype(jnp.int32)
    frac = p - i0.astype(jnp.float32)
    half = jnp.float32(0.5)
    odd = (i0 & 1) == 1
    inc = (frac > half) | ((frac == half) & odd)
    return jnp.where(inc, i0 + 1, i0)


def _body(xs_hbm, ys_hbm, table_hbm, out_hbm, xs_v, ys_v, ids_v, rows_v, sem):
    wid = lax.axis_index("s") * _NC + lax.axis_index("c")
    base = wid * _BPW

    pltpu.sync_copy(xs_hbm.at[pl.ds(base, _BPW)], xs_v)
    pltpu.sync_copy(ys_hbm.at[pl.ds(base, _BPW)], ys_v)

    copies = []
    for c in range(_NCHUNK):
        for jj in range(_CHUNK // 16):
            j = c * (_CHUNK // 16) + jj
            x = xs_v[pl.ds(j * 16, 16)]
            y = ys_v[pl.ds(j * 16, 16)]
            ids_v[pl.ds(j * 16, 16)] = _quantize(x) * _SIDE_NUM + _quantize(y)
        copies.append(pltpu.async_copy(
            table_hbm.at[ids_v.at[pl.ds(c * _CHUNK, _CHUNK)]],
            rows_v.at[pl.ds(c * _CHUNK, _CHUNK)],
            sem))
    for cp in copies:
        cp.wait()

    pltpu.sync_copy(rows_v, out_hbm.at[pl.ds(base, _BPW)])


@jax.jit
def kernel(locations, pos_emb):
    xs = locations[:, 0]
    ys = locations[:, 1]
    mesh = plsc.VectorSubcoreMesh(core_axis_name="c", subcore_axis_name="s")
    run = pl.kernel(
        _body,
        mesh=mesh,
        out_type=jax.ShapeDtypeStruct((_B, _VEC_LEN), jnp.float32),
        scratch_types=[
            pltpu.VMEM((_BPW,), jnp.float32),
            pltpu.VMEM((_BPW,), jnp.float32),
            pltpu.VMEM((_BPW,), jnp.int32),
            pltpu.VMEM((_BPW, _VEC_LEN), jnp.float32),
            pltpu.SemaphoreType.DMA,
        ],
        compiler_params=pltpu.CompilerParams(use_tc_tiling_on_sc=False),
    )
    return run(xs, ys, pos_emb)
